# baseline (speedup 1.0000x reference)
import numpy as np
import jax
import jax.numpy as jnp
from jax import lax
from jax.experimental import pallas as pl
from jax.experimental.pallas import tpu as pltpu

_FEAT_STRIDE = 16.0
_PRE = 2000
_POST = 300
_PAD = 2048
_NB = _PAD // 128
_THR = 0.7
_NA = 9
_SR = 288
_DR = 1152
_TA = _PAD + 128
_HP = lax.Precision.HIGHEST
_BIG = 3.0e38


def _anchor_wh():
    base_size = 16
    ratios = np.array([0.5, 1.0, 2.0])
    scales = np.array([8.0, 16.0, 32.0])
    size = float(base_size) * float(base_size)
    ws = np.round(np.sqrt(size / ratios))
    hs = np.round(ws * ratios)
    aw = np.concatenate([ws[i] * scales for i in range(3)])
    ah = np.concatenate([hs[i] * scales for i in range(3)])
    return aw.astype(np.float32), ah.astype(np.float32)


_AW, _AH = _anchor_wh()


def _iou_gt(rx1, ry1, rx2, ry2, rarea, cx1, cy1, cx2, cy2, carea):
    xx1 = jnp.maximum(rx1, cx1)
    yy1 = jnp.maximum(ry1, cy1)
    xx2 = jnp.minimum(rx2, cx2)
    yy2 = jnp.minimum(ry2, cy2)
    iw = jnp.maximum(xx2 - xx1 + 1.0, 0.0)
    ih = jnp.maximum(yy2 - yy1 + 1.0, 0.0)
    inter = iw * ih
    return inter > _THR * (rarea + carea - inter)


def _split3(v):
    hi = v.astype(jnp.bfloat16).astype(jnp.float32)
    mid = (v - hi).astype(jnp.bfloat16).astype(jnp.float32)
    lo = (v - hi - mid).astype(jnp.bfloat16).astype(jnp.float32)
    return hi, mid, lo


def _onehot_dot(vals, onehot):
    parts = _split3(vals)
    out = None
    for p in parts:
        r = lax.dot_general(onehot, p, (((1,), (1,)), ((), ())),
                            preferred_element_type=jnp.float32)
        out = r if out is None else out + r
    return out


def _body(scores_ref, deltas_ref, imhw_ref, out_ref,
          si_ref, tacc_ref, coords_ref, keep_ref, smat_ref):
    f32 = jnp.float32
    i32 = jnp.int32

    s = scores_ref[0]
    b_ = lax.bitcast_convert_type(s, i32)
    si = jnp.where(b_ >= 0, b_, jnp.int32(0x7FFFFFFF) - b_)
    si_ref[...] = si

    def bs_step(_, lohi):
        lo, hi = lohi
        mid = (lo & hi) + ((lo ^ hi) >> 1)
        mid = mid + ((lo ^ hi) & 1)
        cnt = jnp.sum((si_ref[...] >= mid).astype(f32))
        pred = cnt >= float(_PRE)
        return jnp.where(pred, mid, lo), jnp.where(pred, hi, mid - 1)

    lo0 = jnp.int32(-2147483647) - 1
    hi0 = jnp.int32(2147483647)
    thr, _ = lax.fori_loop(0, 32, bs_step, (lo0, hi0))

    mask = (si_ref[...] >= thr).astype(f32)
    rowcnt = jnp.sum(mask, axis=1, keepdims=True)
    cnt_total = jnp.sum(rowcnt)

    ri = lax.broadcasted_iota(i32, (_SR, _SR), 0)
    rj = lax.broadcasted_iota(i32, (_SR, _SR), 1)
    lstrict288 = jnp.where(rj < ri, 1.0, 0.0)
    offs = lax.dot_general(lstrict288, rowcnt, (((1,), (0,)), ((), ())),
                           preferred_element_type=f32, precision=_HP)

    tacc_ref[...] = jnp.zeros((_TA, 8), f32)

    ii = lax.broadcasted_iota(i32, (128, 128), 0)
    jj = lax.broadcasted_iota(i32, (128, 128), 1)
    eye = jnp.where(ii == jj, 1.0, 0.0)
    tri = jnp.where(ii < jj, 1.0, 0.0)
    lane = lax.broadcasted_iota(i32, (1, 128), 1)
    lane_f = lane.astype(f32)
    band = lax.broadcasted_iota(i32, (128, 1), 0).astype(f32)
    rowsel288 = lax.broadcasted_iota(i32, (_SR, 1), 0)

    def crow(r, _):
        a = r // 32
        rm = r - a * 32
        start = jnp.sum(jnp.where(rowsel288 == r, offs, 0.0))
        start_i = jnp.minimum(start, float(_PAD)).astype(i32)

        srow = scores_ref[0, pl.ds(r, 1), :]
        sirow = si_ref[pl.ds(r, 1), :]
        mrow = (sirow >= thr).astype(f32)
        hwrow = (rm * 128 + lane).astype(f32)
        iflat = hwrow * 9.0 + a.astype(f32)
        drows = [deltas_ref[0, pl.ds(a * 128 + c * 32 + rm, 1), :]
                 for c in range(4)]
        vals8 = jnp.concatenate([srow, iflat] + drows +
                                [jnp.zeros((2, 128), f32)], axis=0)

        pos = lax.dot_general(mrow, tri, (((1,), (0,)), ((), ())),
                              preferred_element_type=f32,
                              precision=_HP) + (start - start_i.astype(f32))
        onehot = jnp.where((pos == band) & (mrow > 0.0), 1.0, 0.0)
        contrib = _onehot_dot(vals8, onehot)
        tacc_ref[pl.ds(start_i, 128), :] = (
            tacc_ref[pl.ds(start_i, 128), :] + contrib)
        return 0

    lax.fori_loop(0, _SR, crow, 0)

    t8cols = []
    for t in range(_NB):
        chunk = tacc_ref[t * 128:(t + 1) * 128, :]
        t8cols.append(lax.dot_general(chunk, eye, (((0,), (0,)), ((), ())),
                                      preferred_element_type=f32,
                                      precision=_HP))
    t8 = jnp.concatenate(t8cols, axis=1)

    qlane = lax.broadcasted_iota(i32, (1, _PAD), 1).astype(f32)
    validt = qlane < cnt_total
    score_t = jnp.where(validt, t8[0:1, :], -_BIG)
    idx_t = jnp.where(validt, t8[1:2, :], _BIG)

    rank = jnp.zeros((1, _PAD), f32)
    for t in range(_NB):
        sc = tacc_ref[t * 128:(t + 1) * 128, 0:1]
        ic = tacc_ref[t * 128:(t + 1) * 128, 1:2]
        qc = (lax.broadcasted_iota(i32, (128, 1), 0) + t * 128).astype(f32)
        vc = qc < cnt_total
        sc = jnp.where(vc, sc, -_BIG)
        ic = jnp.where(vc, ic, _BIG)
        above = jnp.where((sc > score_t) |
                          ((sc == score_t) & (ic < idx_t)), 1.0, 0.0)
        rank = rank + jnp.sum(above, axis=0, keepdims=True)

    srows = []
    for t in range(_NB):
        targ = (lax.broadcasted_iota(i32, (128, 1), 0) + t * 128).astype(f32)
        p2 = jnp.where(rank == targ, 1.0, 0.0)
        srows.append(_onehot_dot(t8, p2))

    def gather_col(c):
        return jnp.concatenate(
            [lax.dot_general(sr[:, c:c + 1], eye, (((0,), (0,)), ((), ())),
                             preferred_element_type=f32, precision=_HP)
             for sr in srows], axis=0)

    idx = gather_col(1)
    dx = gather_col(2)
    dy = gather_col(3)
    dwl = gather_col(4)
    dhl = gather_col(5)

    hw = jnp.floor(idx * (1.0 / 9.0))
    rem = idx - hw * 9.0
    hw = jnp.where(rem >= 9.0, hw + 1.0, jnp.where(rem < 0.0, hw - 1.0, hw))
    a = idx - hw * 9.0
    hpos = jnp.floor(hw * (1.0 / 64.0))
    wpos = hw - hpos * 64.0

    wa = jnp.zeros_like(idx)
    ha = jnp.zeros_like(idx)
    for k in range(_NA):
        wa = jnp.where(a == float(k), float(_AW[k]), wa)
        ha = jnp.where(a == float(k), float(_AH[k]), ha)

    ctr_x = 8.0 + _FEAT_STRIDE * wpos
    ctr_y = 8.0 + _FEAT_STRIDE * hpos
    pcx = dx * wa + ctr_x
    pcy = dy * ha + ctr_y
    pw = jnp.exp(dwl) * wa
    ph = jnp.exp(dhl) * ha

    imh = imhw_ref[0, 0:1, :]
    imw = imhw_ref[0, 1:2, :]
    x1 = jnp.minimum(jnp.maximum(pcx - 0.5 * pw, 0.0), imw - 1.0)
    y1 = jnp.minimum(jnp.maximum(pcy - 0.5 * ph, 0.0), imh - 1.0)
    x2 = jnp.minimum(jnp.maximum(pcx + 0.5 * pw, 0.0), imw - 1.0)
    y2 = jnp.minimum(jnp.maximum(pcy + 0.5 * ph, 0.0), imh - 1.0)
    area = (x2 - x1 + 1.0) * (y2 - y1 + 1.0)

    coords_ref[0] = x1
    coords_ref[1] = y1
    coords_ref[2] = x2
    coords_ref[3] = y2
    coords_ref[4] = area
    keep_ref[...] = jnp.zeros((_NB, 128), f32)

    srange = lax.broadcasted_iota(i32, (304, 1), 0).astype(f32)

    def tpose(m):
        return lax.dot_general(eye, m, (((1,), (1,)), ((), ())),
                               preferred_element_type=f32, precision=_HP)

    def wcond(carry):
        bb, cnt, _ = carry
        return (bb < _NB) & (cnt < float(_POST))

    def wbody(carry):
        bb, cnt, acc = carry
        cols = [coords_ref[k, pl.ds(bb, 1), :] for k in range(5)]

        rows3 = [coords_ref[k].reshape(_NB, 128, 1) for k in range(5)]
        cols3 = [c.reshape(1, 1, 128) for c in cols]
        gt3 = _iou_gt(rows3[0], rows3[1], rows3[2], rows3[3], rows3[4],
                      cols3[0], cols3[1], cols3[2], cols3[3], cols3[4])
        keep3 = keep_ref[...].reshape(_NB, 128, 1)
        supb = jnp.max(jnp.where(gt3, keep3, 0.0), axis=(0, 1)).reshape(1, 128)

        rT = [tpose(c) for c in cols]
        gt = _iou_gt(rT[0], rT[1], rT[2], rT[3], rT[4],
                     cols[0], cols[1], cols[2], cols[3], cols[4])
        smat_ref[...] = jnp.where(gt, tri, 0.0)

        def step(i, sv):
            fi = i.astype(f32)
            kept = 1.0 - jnp.max(jnp.where(lane_f == fi, sv, 0.0))
            row = smat_ref[pl.ds(i, 1), :]
            return jnp.maximum(sv, row * kept)

        supb = lax.fori_loop(0, 128, step, supb, unroll=4)
        keepb = 1.0 - supb

        flatb = bb.astype(f32) * 128.0 + lane_f
        maskedb = jnp.where(flatb < float(_PRE), keepb, 0.0)
        keep_ref[pl.ds(bb, 1), :] = maskedb

        pos = cnt + lax.dot_general(maskedb, tri, (((1,), (0,)), ((), ())),
                                    preferred_element_type=f32, precision=_HP)
        pr = jnp.where((pos == srange) & (maskedb > 0.0), 1.0, 0.0)
        acc = [acc[k] + jnp.sum(pr * cols[k], axis=1, keepdims=True)
               for k in range(4)]
        return bb + 1, cnt + jnp.sum(maskedb), acc

    acc0 = [jnp.zeros((304, 1), f32) for _ in range(4)]
    _, _, acc = lax.while_loop(wcond, wbody, (jnp.int32(0), f32(0.0), acc0))

    bcol = jnp.full((304, 1), 1.0, f32) * pl.program_id(0).astype(f32)
    out5 = jnp.concatenate([bcol] + acc, axis=1)
    out_ref[0] = out5[:_POST, :]


def _run(scores, deltas, imhw, batch):
    return pl.pallas_call(
        _body,
        grid=(batch,),
        in_specs=[
            pl.BlockSpec((1, _SR, 128), lambda b: (b, 0, 0)),
            pl.BlockSpec((1, _DR, 128), lambda b: (b, 0, 0)),
            pl.BlockSpec((1, 8, 128), lambda b: (b, 0, 0)),
        ],
        out_specs=pl.BlockSpec((1, _POST, 5), lambda b: (b, 0, 0)),
        out_shape=jax.ShapeDtypeStruct((batch, _POST, 5), jnp.float32),
        scratch_shapes=[
            pltpu.VMEM((_SR, 128), jnp.int32),
            pltpu.VMEM((_TA, 8), jnp.float32),
            pltpu.VMEM((5, _NB, 128), jnp.float32),
            pltpu.VMEM((_NB, 128), jnp.float32),
            pltpu.VMEM((128, 128), jnp.float32),
        ],
    )(scores, deltas, imhw)


def kernel(cls_score, bbox_deltas, im_shape):
    B = cls_score.shape[0]
    scores = cls_score[:, 1].reshape(B, _SR, 128)
    deltas = bbox_deltas.reshape(B, _DR, 128)
    imhw = jnp.broadcast_to(im_shape[:, :2][:, :, None], (B, 2, 128))
    imhw = jnp.pad(imhw, ((0, 0), (0, 6), (0, 0)))
    return _run(scores, deltas, imhw, B)
